# baseline (speedup 1.0000x reference)
import functools
import numpy as np
import jax
import jax.numpy as jnp
from jax import lax
from jax.experimental import pallas as pl
from jax.experimental.pallas import tpu as pltpu
from jax.experimental.pallas import tpu_sc as plsc

_LORA_DIMS = (8, 16, 32, 48, 64, 96, 128)
_NEXP = len(_LORA_DIMS)
_DMAX = 128
_BLK = 512
_N = 32768
_DIM = 768
_CAP = _N + _NEXP * _BLK
_NBLK = _CAP // _BLK
_NBLKP = 80

_NC, _NS, _L = 2, 16, 16
_NW = _NC * _NS
_TW = _N // _NW
_CH = 128


def _router(x_ref, wg_ref, tril_ref, eid_ref, rank_ref, offx_ref, blk_ref,
            loss_ref, cnt_ref):
    i = pl.program_id(0)
    n = pl.num_programs(0)

    @pl.when(i == 0)
    def _():
        for e in range(_NEXP):
            cnt_ref[e] = 0.0

    x = x_ref[...]
    logits = jax.lax.dot_general(
        x, wg_ref[...], (((1,), (0,)), ((), ())),
        preferred_element_type=jnp.float32)
    amax = jnp.max(logits, axis=1, keepdims=True)
    oh = (logits >= amax).astype(jnp.float32)
    cum = jax.lax.dot_general(
        tril_ref[...], oh, (((1,), (0,)), ((), ())),
        preferred_element_type=jnp.float32)
    lane = jax.lax.broadcasted_iota(jnp.int32, oh.shape, 1)
    base0 = jnp.zeros(oh.shape, jnp.float32)
    for e in range(_NEXP):
        base0 = base0 + jnp.where(lane == e, cnt_ref[e], 0.0)
    rank = jnp.sum(oh * (cum + base0), axis=1)
    eidf = jnp.sum(oh * lane.astype(jnp.float32), axis=1)
    t = x.shape[0]
    eid_ref[...] = eidf.astype(jnp.int32).reshape(1, 1, t)
    rank_ref[...] = rank.astype(jnp.int32).reshape(1, 1, t)
    for e in range(_NEXP):
        cnt_ref[e] += jnp.sum(oh[:, e])

    @pl.when(i == n - 1)
    def _():
        csum = 0.0
        var = 0.0
        for e in range(_NEXP):
            csum += cnt_ref[e]
        mean = csum / _NEXP
        for e in range(_NEXP):
            d = cnt_ref[e] - mean
            var += d * d
        var = var / (_NEXP - 1)
        loss_ref[0, 0] = 0.02 * var / (mean * mean + 1e-10)
        excl = []
        incl = []
        run = jnp.int32(0)
        for e in range(_NEXP):
            ce = cnt_ref[e].astype(jnp.int32)
            pe = ((ce + (_BLK - 1)) // _BLK) * _BLK
            excl.append(run)
            run = run + pe
            incl.append(run)
        for e in range(_NEXP):
            offx_ref[0, e] = excl[e]
        for e in range(_NEXP, 16):
            offx_ref[0, e] = 0
        for b in range(_NBLKP):
            acc = jnp.int32(0)
            for e in range(_NEXP - 1):
                acc += (b * _BLK >= incl[e]).astype(jnp.int32)
            blk_ref[0, b] = acc


def _run_router(x, w_gate):
    n_tok, dim = x.shape
    t1 = 512
    g1 = n_tok // t1
    tril = jnp.asarray(np.tril(np.ones((t1, t1), np.float32), -1))
    return pl.pallas_call(
        _router,
        grid=(g1,),
        in_specs=[
            pl.BlockSpec((t1, dim), lambda i: (i, 0)),
            pl.BlockSpec((dim, _NEXP), lambda i: (0, 0)),
            pl.BlockSpec((t1, t1), lambda i: (0, 0)),
        ],
        out_specs=[
            pl.BlockSpec((1, 1, t1), lambda i: (i, 0, 0)),
            pl.BlockSpec((1, 1, t1), lambda i: (i, 0, 0)),
            pl.BlockSpec(memory_space=pltpu.SMEM, block_shape=(1, 16),
                         index_map=lambda i: (0, 0)),
            pl.BlockSpec(memory_space=pltpu.SMEM, block_shape=(1, _NBLKP),
                         index_map=lambda i: (0, 0)),
            pl.BlockSpec(memory_space=pltpu.SMEM, block_shape=(1, 1),
                         index_map=lambda i: (0, 0)),
        ],
        out_shape=[
            jax.ShapeDtypeStruct((g1, 1, t1), jnp.int32),
            jax.ShapeDtypeStruct((g1, 1, t1), jnp.int32),
            jax.ShapeDtypeStruct((1, 16), jnp.int32),
            jax.ShapeDtypeStruct((1, _NBLKP), jnp.int32),
            jax.ShapeDtypeStruct((1, 1), jnp.float32),
        ],
        scratch_shapes=[pltpu.SMEM((_NEXP,), jnp.float32)],
    )(x, w_gate, tril)


def _dst_body(eid_ref, rank_ref, offx_ref, dst_ref):
    eid = eid_ref[...]
    dst = rank_ref[...]
    for e in range(_NEXP):
        dst = dst + jnp.where(eid == e, offx_ref[0, e], 0)
    dst_ref[...] = dst


def _run_dst(eid3, rank3, offx):
    g1 = eid3.shape[0]
    t1 = eid3.shape[2]
    return pl.pallas_call(
        _dst_body,
        grid=(1,),
        in_specs=[
            pl.BlockSpec((g1, 1, t1), lambda i: (0, 0, 0)),
            pl.BlockSpec((g1, 1, t1), lambda i: (0, 0, 0)),
            pl.BlockSpec(memory_space=pltpu.SMEM, block_shape=(1, 16),
                         index_map=lambda i: (0, 0)),
        ],
        out_specs=pl.BlockSpec((g1, 1, t1), lambda i: (0, 0, 0)),
        out_shape=jax.ShapeDtypeStruct((g1, 1, t1), jnp.int32),
    )(eid3, rank3, offx)


def _sc_dispatch(x_hbm, dst_hbm, xs_hbm, dstbuf, xbuf, sem):
    wid = lax.axis_index("s") * _NC + lax.axis_index("c")
    base = wid * _TW
    for it in range(_TW // _CH):
        b0 = base + it * _CH
        pltpu.sync_copy(dst_hbm.at[pl.ds(b0, _CH)], dstbuf)
        pltpu.sync_copy(x_hbm.at[pl.ds(b0, _CH)], xbuf)
        pltpu.async_copy(xbuf, xs_hbm.at[dstbuf], sem).wait()


def _run_dispatch(x, dst):
    mesh = plsc.VectorSubcoreMesh(core_axis_name="c", subcore_axis_name="s")
    fn = functools.partial(
        pl.kernel, mesh=mesh,
        out_type=jax.ShapeDtypeStruct((_CAP, _DIM), jnp.float32),
        scratch_types=[
            pltpu.VMEM((_CH,), jnp.int32),
            pltpu.VMEM((_CH, _DIM), jnp.float32),
            pltpu.SemaphoreType.DMA,
        ],
    )(_sc_dispatch)
    return fn(x, dst)


def _mm_body(bm_ref, xs_ref, at3_ref, bt3_ref, ys_ref):
    h = jax.lax.dot_general(
        xs_ref[...], at3_ref[0], (((1,), (0,)), ((), ())),
        preferred_element_type=jnp.float32)
    ys_ref[...] = jax.lax.dot_general(
        h, bt3_ref[0], (((1,), (0,)), ((), ())),
        preferred_element_type=jnp.float32)


def _run_mm(blkmap, xs, at3, bt3):
    grid_spec = pltpu.PrefetchScalarGridSpec(
        num_scalar_prefetch=1,
        grid=(_NBLK,),
        in_specs=[
            pl.BlockSpec((_BLK, _DIM), lambda i, bm: (i, 0)),
            pl.BlockSpec((1, _DIM, _DMAX), lambda i, bm: (bm[i], 0, 0)),
            pl.BlockSpec((1, _DMAX, _DIM), lambda i, bm: (bm[i], 0, 0)),
        ],
        out_specs=pl.BlockSpec((_BLK, _DIM), lambda i, bm: (i, 0)),
    )
    return pl.pallas_call(
        _mm_body,
        grid_spec=grid_spec,
        out_shape=jax.ShapeDtypeStruct((_CAP, _DIM), jnp.float32),
    )(blkmap, xs, at3, bt3)


def _sc_combine(ys_hbm, dst_hbm, y_hbm, dstbuf, ybuf, sem):
    wid = lax.axis_index("s") * _NC + lax.axis_index("c")
    base = wid * _TW
    for it in range(_TW // _CH):
        b0 = base + it * _CH
        pltpu.sync_copy(dst_hbm.at[pl.ds(b0, _CH)], dstbuf)
        pltpu.async_copy(ys_hbm.at[dstbuf], ybuf, sem).wait()
        pltpu.sync_copy(ybuf, y_hbm.at[pl.ds(b0, _CH)])


def _run_combine(ys, dst):
    mesh = plsc.VectorSubcoreMesh(core_axis_name="c", subcore_axis_name="s")
    fn = functools.partial(
        pl.kernel, mesh=mesh,
        out_type=jax.ShapeDtypeStruct((_N, _DIM), jnp.float32),
        scratch_types=[
            pltpu.VMEM((_CH,), jnp.int32),
            pltpu.VMEM((_CH, _DIM), jnp.float32),
            pltpu.SemaphoreType.DMA,
        ],
    )(_sc_combine)
    return fn(ys, dst)


def kernel(x, w_gate, A0, B0, A1, B1, A2, B2, A3, B3, A4, B4, A5, B5, A6, B6):
    As = (A0, A1, A2, A3, A4, A5, A6)
    Bs = (B0, B1, B2, B3, B4, B5, B6)
    at3 = jnp.stack([jnp.pad(a.T, ((0, 0), (0, _DMAX - a.shape[0])))
                     for a in As])
    bt3 = jnp.stack([jnp.pad(b.T, ((0, _DMAX - b.shape[1]), (0, 0)))
                     for b in Bs])

    eid3, rank3, offx, blkmap, loss = _run_router(x, w_gate)
    dst = _run_dst(eid3, rank3, offx).reshape(_N)
    blkmap = blkmap.reshape(_NBLKP)

    xs = _run_dispatch(x, dst)
    ys = _run_mm(blkmap, xs, at3, bt3)
    y = _run_combine(ys, dst)
    return y, loss[0, 0]


# speedup vs baseline: 3.6049x; 3.6049x over previous
import numpy as np
import jax
import jax.numpy as jnp
from jax.experimental import pallas as pl
from jax.experimental.pallas import tpu as pltpu

_LORA_DIMS = (8, 16, 32, 48, 64, 96, 128)
_NEXP = len(_LORA_DIMS)
_DSUM = sum(_LORA_DIMS)
_DPAD = 512
_STARTS = tuple(np.cumsum((0,) + _LORA_DIMS).tolist())


def _expmap():
    m = np.zeros((_NEXP, _DPAD), np.float32)
    for e in range(_NEXP):
        m[e, _STARTS[e]:_STARTS[e + 1]] = 1.0
    return m


def _body(x_ref, wg_ref, at_ref, bt_ref, em_ref, y_ref, loss_ref, cnt_ref):
    i = pl.program_id(0)
    n = pl.num_programs(0)
    x = x_ref[...]

    logits = jax.lax.dot_general(
        x, wg_ref[...], (((1,), (0,)), ((), ())),
        preferred_element_type=jnp.float32)
    amax = jnp.max(logits, axis=1, keepdims=True)
    oh = (logits >= amax).astype(jnp.float32)

    @pl.when(i == 0)
    def _():
        cnt_ref[...] = jnp.zeros_like(cnt_ref)

    cnt_ref[...] += jnp.sum(oh, axis=0, keepdims=True)

    h = jax.lax.dot_general(
        x, at_ref[...], (((1,), (0,)), ((), ())),
        preferred_element_type=jnp.float32)
    sel = jax.lax.dot_general(
        oh, em_ref[...], (((1,), (0,)), ((), ())),
        preferred_element_type=jnp.float32)
    h = h * sel

    o = jax.lax.dot_general(
        h, bt_ref[...], (((1,), (0,)), ((), ())),
        preferred_element_type=jnp.float32)
    y_ref[...] = o

    @pl.when(i == n - 1)
    def _():
        c = cnt_ref[0, :]
        csum = jnp.sum(c)
        mean = csum / _NEXP
        var = jnp.sum((c - mean) * (c - mean)) / (_NEXP - 1)
        loss_ref[0, 0] = 0.02 * var / (mean * mean + 1e-10)


def kernel(x, w_gate, A0, B0, A1, B1, A2, B2, A3, B3, A4, B4, A5, B5, A6, B6):
    As = (A0, A1, A2, A3, A4, A5, A6)
    Bs = (B0, B1, B2, B3, B4, B5, B6)
    n_tok, dim = x.shape
    at = jnp.concatenate([a.T for a in As], axis=1)
    at = jnp.pad(at, ((0, 0), (0, _DPAD - _DSUM)))
    bt = jnp.concatenate([b.T for b in Bs], axis=0)
    bt = jnp.pad(bt, ((0, _DPAD - _DSUM), (0, 0)))
    em = jnp.asarray(_expmap())

    tile = 2048
    grid = n_tok // tile

    y, loss = pl.pallas_call(
        _body,
        grid=(grid,),
        in_specs=[
            pl.BlockSpec((tile, dim), lambda i: (i, 0)),
            pl.BlockSpec((dim, _NEXP), lambda i: (0, 0)),
            pl.BlockSpec((dim, _DPAD), lambda i: (0, 0)),
            pl.BlockSpec((_DPAD, dim), lambda i: (0, 0)),
            pl.BlockSpec((_NEXP, _DPAD), lambda i: (0, 0)),
        ],
        out_specs=[
            pl.BlockSpec((tile, dim), lambda i: (i, 0)),
            pl.BlockSpec(memory_space=pltpu.SMEM, block_shape=(1, 1),
                         index_map=lambda i: (0, 0)),
        ],
        out_shape=[
            jax.ShapeDtypeStruct((n_tok, dim), jnp.float32),
            jax.ShapeDtypeStruct((1, 1), jnp.float32),
        ],
        scratch_shapes=[pltpu.VMEM((1, _NEXP), jnp.float32)],
    )(x, w_gate, at, bt, em)
    return y, loss[0, 0]
